# baseline (speedup 1.0000x reference)
import functools

import numpy as np
import jax
import jax.numpy as jnp
from jax.experimental import pallas as pl
from jax.experimental.pallas import tpu as pltpu
from jax.experimental.pallas import tpu_sc as plsc

B = 8
N = 2048
K = 64
NWORKERS = 32
LANES = 16


def _fps_kernel(x_ref, y_ref, z_ref, qx_ref, qy_ref, qz_ref):
    P = x_ref.shape[1]
    S = qx_ref.shape[1]
    x = x_ref[...]
    y = y_ref[...]
    z = z_ref[...]
    iota = jax.lax.broadcasted_iota(jnp.int32, (B, P), 1)
    dx = x - x[:, 0:1]
    dy = y - y[:, 0:1]
    dz = z - z[:, 0:1]
    min_d = dx * dx + dy * dy + dz * dz
    iota_s = jax.lax.broadcasted_iota(jnp.int32, (B, S), 1)
    qx0 = jnp.broadcast_to(x[:, 0:1], (B, S))
    qy0 = jnp.broadcast_to(y[:, 0:1], (B, S))
    qz0 = jnp.broadcast_to(z[:, 0:1], (B, S))

    def body(i, carry):
        min_d, qx, qy, qz = carry
        m = jnp.max(min_d, axis=1, keepdims=True)
        is_max = min_d == m
        idx = jnp.min(jnp.where(is_max, iota, P), axis=1, keepdims=True)
        onehot = iota == idx
        px = jnp.sum(jnp.where(onehot, x, 0.0), axis=1, keepdims=True)
        py = jnp.sum(jnp.where(onehot, y, 0.0), axis=1, keepdims=True)
        pz = jnp.sum(jnp.where(onehot, z, 0.0), axis=1, keepdims=True)
        lane = iota_s == i
        qx = jnp.where(lane, px, qx)
        qy = jnp.where(lane, py, qy)
        qz = jnp.where(lane, pz, qz)
        ddx = x - px
        ddy = y - py
        ddz = z - pz
        d = ddx * ddx + ddy * ddy + ddz * ddz
        return (jnp.minimum(min_d, d), qx, qy, qz)

    _, qx, qy, qz = jax.lax.fori_loop(1, S, body, (min_d, qx0, qy0, qz0))
    qx_ref[...] = qx
    qy_ref[...] = qy
    qz_ref[...] = qz


def _fps_pallas(x, y, z, S):
    return pl.pallas_call(
        _fps_kernel,
        out_shape=[jax.ShapeDtypeStruct((B, S), jnp.float32)] * 3,
    )(x, y, z)


def _bisect_kernel(r2bits, qx_ref, qy_ref, qz_ref, px_ref, py_ref, pz_ref,
                   d2_ref, tb_ref, qbx_ref, qby_ref, qbz_ref):
    QB = qx_ref.shape[0]
    qxc = qx_ref[...]
    qyc = qy_ref[...]
    qzc = qz_ref[...]
    dx = qxc - px_ref[0]
    dy = qyc - py_ref[0]
    dz = qzc - pz_ref[0]
    d2 = dx * dx + dy * dy + dz * dz
    d2_ref[...] = d2
    qbx_ref[...] = jnp.broadcast_to(qxc, (QB, LANES))
    qby_ref[...] = jnp.broadcast_to(qyc, (QB, LANES))
    qbz_ref[...] = jnp.broadcast_to(qzc, (QB, LANES))
    bits = jax.lax.bitcast_convert_type(d2, jnp.int32)
    lo = jnp.full((QB, 1), -1, jnp.int32)
    hi = jnp.full((QB, 1), r2bits, jnp.int32)

    def body(_, carry):
        lo, hi = carry
        mid = lo + ((hi - lo) >> 1)
        cnt = jnp.sum(jnp.where(bits <= mid, 1, 0), axis=1, keepdims=True)
        ge = cnt >= K
        return (jnp.where(ge, lo, mid), jnp.where(ge, mid, hi))

    _, hi = jax.lax.fori_loop(0, 31, body, (lo, hi))
    t = jax.lax.bitcast_convert_type(hi, jnp.float32)
    tb_ref[...] = jnp.broadcast_to(t, (QB, LANES))


def _bisect_pallas(qx, qy, qz, px, py, pz, r):
    S = qx.shape[1]
    P = px.shape[1]
    NQ = B * S
    QB = 256
    grid = NQ // QB
    blocks_per_cloud = S // QB
    r2bits = int(np.float32(r * r).view(np.int32))
    qxT = qx.reshape(NQ, 1)
    qyT = qy.reshape(NQ, 1)
    qzT = qz.reshape(NQ, 1)
    qspec = pl.BlockSpec((QB, 1), lambda g: (g, 0))
    pspec = pl.BlockSpec((1, 1, P), lambda g: (g // blocks_per_cloud, 0, 0))
    bspec = pl.BlockSpec((QB, LANES), lambda g: (g, 0))
    bshape = jax.ShapeDtypeStruct((NQ, LANES), jnp.float32)
    px = px.reshape(B, 1, P)
    py = py.reshape(B, 1, P)
    pz = pz.reshape(B, 1, P)
    d2, tb, qbx, qby, qbz = pl.pallas_call(
        functools.partial(_bisect_kernel, r2bits),
        grid=(grid,),
        in_specs=[qspec, qspec, qspec, pspec, pspec, pspec],
        out_specs=[pl.BlockSpec((QB, P), lambda g: (g, 0)),
                   bspec, bspec, bspec, bspec],
        out_shape=[jax.ShapeDtypeStruct((NQ, P), jnp.float32),
                   bshape, bshape, bshape, bshape],
    )(qxT, qyT, qzT, px, py, pz)
    return d2, tb, qbx, qby, qbz


def _compact_kernel(P, QPW, S1, with_x, refs):
    if with_x:
        (d2_hbm, tb_hbm, qbx_hbm, qby_hbm, qbz_hbm, px_hbm, py_hbm, pz_hbm,
         x1_hbm, rx_hbm, ry_hbm, rz_hbm, nbrx_hbm,
         d2a_v, d2b_v, tb_v, qbx_v, qby_v, qbz_v, px_v, py_v, pz_v,
         idx_v, rbx_v, rby_v, rbz_v, gidx_v, xrow_v,
         sema, semb, semg) = refs
    else:
        (d2_hbm, tb_hbm, qbx_hbm, qby_hbm, qbz_hbm, px_hbm, py_hbm, pz_hbm,
         rx_hbm, ry_hbm, rz_hbm,
         d2a_v, d2b_v, tb_v, qbx_v, qby_v, qbz_v, px_v, py_v, pz_v,
         idx_v, rbx_v, rby_v, rbz_v,
         sema, semb) = refs
    STEPS = P // LANES
    QB = 16
    wid = jax.lax.axis_index("s") * 2 + jax.lax.axis_index("c")
    qbase = wid * QPW
    cloud = qbase // S1
    iota = jax.lax.broadcasted_iota(jnp.int32, (LANES,), 0)

    pltpu.sync_copy(tb_hbm.at[pl.ds(qbase * LANES, QPW * LANES)], tb_v)
    pltpu.sync_copy(qbx_hbm.at[pl.ds(qbase * LANES, QPW * LANES)], qbx_v)
    pltpu.sync_copy(qby_hbm.at[pl.ds(qbase * LANES, QPW * LANES)], qby_v)
    pltpu.sync_copy(qbz_hbm.at[pl.ds(qbase * LANES, QPW * LANES)], qbz_v)
    pltpu.sync_copy(px_hbm.at[pl.ds(cloud * P, P)], px_v)
    pltpu.sync_copy(py_hbm.at[pl.ds(cloud * P, P)], py_v)
    pltpu.sync_copy(pz_hbm.at[pl.ds(cloud * P, P)], pz_v)
    pltpu.make_async_copy(d2_hbm.at[pl.ds(qbase * P, P)], d2a_v, sema).start()

    def process_one(i, j, d2buf):
        tv = plsc.load_gather(tb_v, [i * LANES + iota])

        def step(s, carry):
            cnt_m1, iv = carry
            d2v = plsc.load_gather(d2buf, [iv])
            m = d2v <= tv
            ranks = plsc.cumsum(jnp.where(m, 1, 0))
            tgt = cnt_m1 + ranks
            keep = jnp.logical_and(m, tgt <= K - 1)
            plsc.store_scatter(idx_v, [tgt + (j * K)], iv, mask=keep)
            pc = plsc.all_reduce_population_count(keep)
            return (cnt_m1 + pc, iv + LANES)

        cnt_m1, _ = jax.lax.fori_loop(
            0, STEPS, step,
            (jnp.full((LANES,), -1, jnp.int32), iota), unroll=4)
        v0 = idx_v[pl.ds(j * K, LANES)]
        pad = plsc.cummax(jnp.where(iota == 0, v0, jnp.int32(-(2 ** 31))))
        for k in range(K // LANES):
            lanes = iota + (16 * k)
            plsc.store_scatter(idx_v, [lanes + (j * K)], pad,
                               mask=lanes > cnt_m1)
        qxv = plsc.load_gather(qbx_v, [i * LANES + iota])
        qyv = plsc.load_gather(qby_v, [i * LANES + iota])
        qzv = plsc.load_gather(qbz_v, [i * LANES + iota])
        for k in range(K // LANES):
            iv = idx_v[pl.ds(j * K + 16 * k, LANES)]
            rbx_v[pl.ds(j * K + 16 * k, LANES)] = (
                plsc.load_gather(px_v, [iv]) - qxv)
            rby_v[pl.ds(j * K + 16 * k, LANES)] = (
                plsc.load_gather(py_v, [iv]) - qyv)
            rbz_v[pl.ds(j * K + 16 * k, LANES)] = (
                plsc.load_gather(pz_v, [iv]) - qzv)
        if with_x:
            for k in range(K // LANES):
                iv = idx_v[pl.ds(j * K + 16 * k, LANES)]
                gidx_v[pl.ds(16 * k, LANES)] = iv + cloud * P
            gdesc = pltpu.make_async_copy(x1_hbm.at[gidx_v], xrow_v, semg)
            gdesc.start()
            gdesc.wait()
            pltpu.sync_copy(
                xrow_v, nbrx_hbm.at[pl.ds((qbase + i) * K, K)])

    nbatch = QPW // QB

    def batch_body(b, _):
        for j in range(QB):
            i = b * QB + j
            buf_cur = d2a_v if j % 2 == 0 else d2b_v
            buf_nxt = d2b_v if j % 2 == 0 else d2a_v
            sem_cur = sema if j % 2 == 0 else semb
            sem_nxt = semb if j % 2 == 0 else sema
            pltpu.make_async_copy(
                d2_hbm.at[pl.ds(qbase * P, P)], buf_cur, sem_cur).wait()

            @pl.when(i + 1 < QPW)
            def _():
                pltpu.make_async_copy(
                    d2_hbm.at[pl.ds((qbase + i + 1) * P, P)],
                    buf_nxt, sem_nxt).start()

            process_one(i, j, buf_cur)
        base = (qbase + b * QB) * K
        pltpu.sync_copy(rbx_v, rx_hbm.at[pl.ds(base, QB * K)])
        pltpu.sync_copy(rby_v, ry_hbm.at[pl.ds(base, QB * K)])
        pltpu.sync_copy(rbz_v, rz_hbm.at[pl.ds(base, QB * K)])
        return 0

    jax.lax.fori_loop(0, nbatch, batch_body, 0)


def _compact_pallas(d2, tb, qbx, qby, qbz, px, py, pz, x1=None):
    NQ, P = d2.shape
    S1 = NQ // B
    QPW = NQ // NWORKERS
    with_x = x1 is not None
    mesh = plsc.VectorSubcoreMesh(core_axis_name="c", subcore_axis_name="s")
    relshape = jax.ShapeDtypeStruct((NQ * K,), jnp.float32)
    out_type = [relshape, relshape, relshape]
    scratch = [
        pltpu.VMEM((P,), jnp.float32),
        pltpu.VMEM((P,), jnp.float32),
        pltpu.VMEM((QPW * LANES,), jnp.float32),
        pltpu.VMEM((QPW * LANES,), jnp.float32),
        pltpu.VMEM((QPW * LANES,), jnp.float32),
        pltpu.VMEM((QPW * LANES,), jnp.float32),
        pltpu.VMEM((P,), jnp.float32),
        pltpu.VMEM((P,), jnp.float32),
        pltpu.VMEM((P,), jnp.float32),
        pltpu.VMEM((16 * K,), jnp.int32),
        pltpu.VMEM((16 * K,), jnp.float32),
        pltpu.VMEM((16 * K,), jnp.float32),
        pltpu.VMEM((16 * K,), jnp.float32),
    ]
    if with_x:
        XF = x1.shape[1]
        out_type = out_type + [
            jax.ShapeDtypeStruct((NQ * K, XF), jnp.float32)]
        scratch = scratch + [
            pltpu.VMEM((K,), jnp.int32),
            pltpu.VMEM((K, XF), jnp.float32),
            pltpu.SemaphoreType.DMA,
        ]
    scratch = scratch + [pltpu.SemaphoreType.DMA, pltpu.SemaphoreType.DMA]

    def body(*refs):
        _compact_kernel(P, QPW, S1, with_x, refs)

    kfn = pl.kernel(
        body,
        out_type=out_type,
        mesh=mesh,
        compiler_params=pltpu.CompilerParams(needs_layout_passes=False),
        scratch_types=scratch,
    )
    args = [d2.reshape(NQ * P), tb.reshape(NQ * LANES),
            qbx.reshape(NQ * LANES), qby.reshape(NQ * LANES),
            qbz.reshape(NQ * LANES),
            px.reshape(B * P), py.reshape(B * P), pz.reshape(B * P)]
    if with_x:
        args.append(x1)
        rx, ry, rz, nbrx = kfn(*args)
        return (rx.reshape(NQ, K), ry.reshape(NQ, K), rz.reshape(NQ, K),
                nbrx)
    rx, ry, rz = kfn(*args)
    return rx.reshape(NQ, K), ry.reshape(NQ, K), rz.reshape(NQ, K), None


def _mlp1_kernel(rx_ref, ry_ref, rz_ref, w1_ref, b1_ref, w2_ref, b2_ref,
                 out_ref):
    QB = rx_ref.shape[0]
    F1 = w1_ref.shape[1]
    F2 = out_ref.shape[1]

    def pairs(ref):
        v = ref[...][:, :, None]
        return jnp.broadcast_to(v, (QB, K, F1)).reshape(QB * K, F1)

    h = (pairs(rx_ref) * w1_ref[0:1, :] + pairs(ry_ref) * w1_ref[1:2, :]
         + pairs(rz_ref) * w1_ref[2:3, :] + b1_ref[...])
    h = jnp.maximum(h, 0.0)
    h = jnp.maximum(h @ w2_ref[...] + b2_ref[...], 0.0)
    x = jnp.max(h.reshape(QB, K, w2_ref.shape[1]), axis=1)
    out_ref[...] = jnp.concatenate(
        [x, jnp.zeros((QB, F2 - x.shape[1]), jnp.float32)], axis=1)


def _mlp1_pallas(rx, ry, rz, W11, b11, W12, b12):
    NQ = rx.shape[0]
    QB = 128
    grid = NQ // QB
    rspec = pl.BlockSpec((QB, K), lambda g: (g, 0))
    F1 = W12.shape[0]
    F2 = 2 * W12.shape[1]
    x1 = pl.pallas_call(
        _mlp1_kernel,
        grid=(grid,),
        in_specs=[rspec, rspec, rspec,
                  pl.BlockSpec((3, F1), lambda g: (0, 0)),
                  pl.BlockSpec((1, F1), lambda g: (0, 0)),
                  pl.BlockSpec((F1, W12.shape[1]), lambda g: (0, 0)),
                  pl.BlockSpec((1, W12.shape[1]), lambda g: (0, 0))],
        out_specs=pl.BlockSpec((QB, F2), lambda g: (g, 0)),
        out_shape=jax.ShapeDtypeStruct((NQ, F2), jnp.float32),
    )(rx, ry, rz, W11, b11[None, :], W12, b12[None, :])
    return x1


def _mlp2g_kernel(nbrx_ref, rx_ref, ry_ref, rz_ref, qx_ref, qy_ref, qz_ref,
                  w21a_ref, w21r_ref, b21_ref, w22_ref, b22_ref,
                  wg1_ref, bg1_ref, wg2_ref, bg2_ref, out_ref, *, bpc):
    QB = rx_ref.shape[0]
    F1 = w21a_ref.shape[1]

    def pairs(ref, width):
        v = ref[...][:, :, None]
        return jnp.broadcast_to(v, (QB, K, width)).reshape(QB * K, width)

    h = (nbrx_ref[...] @ w21a_ref[...]
         + pairs(rx_ref, F1) * w21r_ref[0:1, :]
         + pairs(ry_ref, F1) * w21r_ref[1:2, :]
         + pairs(rz_ref, F1) * w21r_ref[2:3, :]
         + b21_ref[...])
    h = jnp.maximum(h, 0.0)
    h = jnp.maximum(h @ w22_ref[...] + b22_ref[...], 0.0)
    F2 = w22_ref.shape[1]
    x2 = jnp.max(h.reshape(QB, K, F2), axis=1)
    g = (x2 @ wg1_ref[0:F2, :]
         + qx_ref[...] * wg1_ref[F2:F2 + 1, :]
         + qy_ref[...] * wg1_ref[F2 + 1:F2 + 2, :]
         + qz_ref[...] * wg1_ref[F2 + 2:F2 + 3, :]
         + bg1_ref[...])
    g = jnp.maximum(g, 0.0)
    g = g @ wg2_ref[...] + bg2_ref[...]
    m = jnp.max(g, axis=0, keepdims=True)[None]
    gi = pl.program_id(0)

    @pl.when(gi % bpc == 0)
    def _():
        out_ref[...] = m

    @pl.when(gi % bpc != 0)
    def _():
        out_ref[...] = jnp.maximum(out_ref[...], m)


def _mlp2g_pallas(nbrx, rx, ry, rz, q2x, q2y, q2z,
                  W21a, W21r, b21, W22, b22, Wg1, bg1, Wg2, bg2):
    NQ = rx.shape[0]
    S2 = NQ // B
    QB = 64
    grid = NQ // QB
    bpc = S2 // QB
    rspec = pl.BlockSpec((QB, K), lambda g: (g, 0))
    qspec = pl.BlockSpec((QB, 1), lambda g: (g, 0))
    XF = W21a.shape[0]
    F1 = W21a.shape[1]
    F2 = W22.shape[1]
    G1 = Wg1.shape[1]
    G2 = Wg2.shape[1]
    out = pl.pallas_call(
        functools.partial(_mlp2g_kernel, bpc=bpc),
        grid=(grid,),
        in_specs=[pl.BlockSpec((QB * K, XF), lambda g: (g, 0)),
                  rspec, rspec, rspec, qspec, qspec, qspec,
                  pl.BlockSpec((XF, F1), lambda g: (0, 0)),
                  pl.BlockSpec((3, F1), lambda g: (0, 0)),
                  pl.BlockSpec((1, F1), lambda g: (0, 0)),
                  pl.BlockSpec((F1, F2), lambda g: (0, 0)),
                  pl.BlockSpec((1, F2), lambda g: (0, 0)),
                  pl.BlockSpec((F2 + 3, G1), lambda g: (0, 0)),
                  pl.BlockSpec((1, G1), lambda g: (0, 0)),
                  pl.BlockSpec((G1, G2), lambda g: (0, 0)),
                  pl.BlockSpec((1, G2), lambda g: (0, 0))],
        out_specs=pl.BlockSpec((1, 1, G2), lambda g: (g // bpc, 0, 0)),
        out_shape=jax.ShapeDtypeStruct((B, 1, G2), jnp.float32),
    )(nbrx, rx, ry, rz, q2x.reshape(NQ, 1), q2y.reshape(NQ, 1),
      q2z.reshape(NQ, 1), W21a, W21r, b21[None, :], W22, b22[None, :],
      Wg1, bg1[None, :], Wg2, bg2[None, :])
    return out.reshape(B, G2)


def kernel(pos, batch, W11, b11, W12, b12, W21, b21, W22, b22, Wg1, bg1, Wg2, bg2):
    pos3 = pos.reshape(B, N, 3)
    px, py, pz = pos3[:, :, 0], pos3[:, :, 1], pos3[:, :, 2]
    W21a = jnp.concatenate([W21[:K], jnp.zeros_like(W21[:K])], axis=0)
    W21r = W21[K:]
    qx, qy, qz = _fps_pallas(px, py, pz, N // 2)
    d2, tb, qbx, qby, qbz = _bisect_pallas(qx, qy, qz, px, py, pz, 0.2)
    rx, ry, rz, _ = _compact_pallas(d2, tb, qbx, qby, qbz, px, py, pz)
    x1 = _mlp1_pallas(rx, ry, rz, W11, b11, W12, b12)
    q2x, q2y, q2z = _fps_pallas(qx, qy, qz, N // 8)
    d2b, tbb, qbx2, qby2, qbz2 = _bisect_pallas(q2x, q2y, q2z, qx, qy, qz, 0.4)
    rx2, ry2, rz2, nbrx = _compact_pallas(
        d2b, tbb, qbx2, qby2, qbz2, qx, qy, qz, x1)
    return _mlp2g_pallas(nbrx, rx2, ry2, rz2, q2x, q2y, q2z,
                         W21a, W21r, b21, W22, b22, Wg1, bg1, Wg2, bg2)


# speedup vs baseline: 1.0034x; 1.0034x over previous
import functools

import numpy as np
import jax
import jax.numpy as jnp
from jax.experimental import pallas as pl
from jax.experimental.pallas import tpu as pltpu
from jax.experimental.pallas import tpu_sc as plsc

B = 8
N = 2048
K = 64
NWORKERS = 32
LANES = 16


def _fps_kernel(x_ref, y_ref, z_ref, qx_ref, qy_ref, qz_ref):
    P = x_ref.shape[1]
    S = qx_ref.shape[1]
    x = x_ref[...]
    y = y_ref[...]
    z = z_ref[...]
    iota = jax.lax.broadcasted_iota(jnp.int32, (B, P), 1)
    dx = x - x[:, 0:1]
    dy = y - y[:, 0:1]
    dz = z - z[:, 0:1]
    min_d = dx * dx + dy * dy + dz * dz
    iota_s = jax.lax.broadcasted_iota(jnp.int32, (B, S), 1)
    qx0 = jnp.broadcast_to(x[:, 0:1], (B, S))
    qy0 = jnp.broadcast_to(y[:, 0:1], (B, S))
    qz0 = jnp.broadcast_to(z[:, 0:1], (B, S))

    def body(i, carry):
        min_d, qx, qy, qz = carry
        m = jnp.max(min_d, axis=1, keepdims=True)
        is_max = min_d == m
        idx = jnp.min(jnp.where(is_max, iota, P), axis=1, keepdims=True)
        onehot = iota == idx
        px = jnp.sum(jnp.where(onehot, x, 0.0), axis=1, keepdims=True)
        py = jnp.sum(jnp.where(onehot, y, 0.0), axis=1, keepdims=True)
        pz = jnp.sum(jnp.where(onehot, z, 0.0), axis=1, keepdims=True)
        lane = iota_s == i
        qx = jnp.where(lane, px, qx)
        qy = jnp.where(lane, py, qy)
        qz = jnp.where(lane, pz, qz)
        ddx = x - px
        ddy = y - py
        ddz = z - pz
        d = ddx * ddx + ddy * ddy + ddz * ddz
        return (jnp.minimum(min_d, d), qx, qy, qz)

    _, qx, qy, qz = jax.lax.fori_loop(1, S, body, (min_d, qx0, qy0, qz0))
    qx_ref[...] = qx
    qy_ref[...] = qy
    qz_ref[...] = qz


def _fps_pallas(x, y, z, S):
    return pl.pallas_call(
        _fps_kernel,
        out_shape=[jax.ShapeDtypeStruct((B, S), jnp.float32)] * 3,
    )(x, y, z)


def _bisect_kernel(r2bits, qx_ref, qy_ref, qz_ref, px_ref, py_ref, pz_ref,
                   d2_ref, tb_ref, qbx_ref, qby_ref, qbz_ref):
    QB = qx_ref.shape[0]
    qxc = qx_ref[...]
    qyc = qy_ref[...]
    qzc = qz_ref[...]
    dx = qxc - px_ref[0]
    dy = qyc - py_ref[0]
    dz = qzc - pz_ref[0]
    d2 = dx * dx + dy * dy + dz * dz
    d2_ref[...] = d2
    qbx_ref[...] = jnp.broadcast_to(qxc, (QB, LANES))
    qby_ref[...] = jnp.broadcast_to(qyc, (QB, LANES))
    qbz_ref[...] = jnp.broadcast_to(qzc, (QB, LANES))
    bits = jax.lax.bitcast_convert_type(d2, jnp.int32)
    lo = jnp.full((QB, 1), -1, jnp.int32)
    hi = jnp.full((QB, 1), r2bits, jnp.int32)

    def body(_, carry):
        lo, hi = carry
        mid = lo + ((hi - lo) >> 1)
        cnt = jnp.sum(jnp.where(bits <= mid, 1, 0), axis=1, keepdims=True)
        ge = cnt >= K
        return (jnp.where(ge, lo, mid), jnp.where(ge, mid, hi))

    _, hi = jax.lax.fori_loop(0, 31, body, (lo, hi))
    t = jax.lax.bitcast_convert_type(hi, jnp.float32)
    tb_ref[...] = jnp.broadcast_to(t, (QB, LANES))


def _bisect_pallas(qx, qy, qz, px, py, pz, r):
    S = qx.shape[1]
    P = px.shape[1]
    NQ = B * S
    QB = 256
    grid = NQ // QB
    blocks_per_cloud = S // QB
    r2bits = int(np.float32(r * r).view(np.int32))
    qxT = qx.reshape(NQ, 1)
    qyT = qy.reshape(NQ, 1)
    qzT = qz.reshape(NQ, 1)
    qspec = pl.BlockSpec((QB, 1), lambda g: (g, 0))
    pspec = pl.BlockSpec((1, 1, P), lambda g: (g // blocks_per_cloud, 0, 0))
    bspec = pl.BlockSpec((QB, LANES), lambda g: (g, 0))
    bshape = jax.ShapeDtypeStruct((NQ, LANES), jnp.float32)
    px = px.reshape(B, 1, P)
    py = py.reshape(B, 1, P)
    pz = pz.reshape(B, 1, P)
    d2, tb, qbx, qby, qbz = pl.pallas_call(
        functools.partial(_bisect_kernel, r2bits),
        grid=(grid,),
        in_specs=[qspec, qspec, qspec, pspec, pspec, pspec],
        out_specs=[pl.BlockSpec((QB, P), lambda g: (g, 0)),
                   bspec, bspec, bspec, bspec],
        out_shape=[jax.ShapeDtypeStruct((NQ, P), jnp.float32),
                   bshape, bshape, bshape, bshape],
    )(qxT, qyT, qzT, px, py, pz)
    return d2, tb, qbx, qby, qbz


def _compact_kernel(P, QPW, S1, with_x, refs):
    if with_x:
        (d2_hbm, tb_hbm, qbx_hbm, qby_hbm, qbz_hbm, px_hbm, py_hbm, pz_hbm,
         x1_hbm, rx_hbm, ry_hbm, rz_hbm, nbrx_hbm,
         d2a_v, d2b_v, tb_v, qbx_v, qby_v, qbz_v, px_v, py_v, pz_v,
         idx_v, rbx_v, rby_v, rbz_v, gidx_v, xrow_v,
         sema, semb, semg) = refs
    else:
        (d2_hbm, tb_hbm, qbx_hbm, qby_hbm, qbz_hbm, px_hbm, py_hbm, pz_hbm,
         rx_hbm, ry_hbm, rz_hbm,
         d2a_v, d2b_v, tb_v, qbx_v, qby_v, qbz_v, px_v, py_v, pz_v,
         idx_v, rbx_v, rby_v, rbz_v,
         sema, semb) = refs
    STEPS = P // LANES
    QB = 16
    wid = jax.lax.axis_index("s") * 2 + jax.lax.axis_index("c")
    qbase = wid * QPW
    cloud = qbase // S1
    iota = jax.lax.broadcasted_iota(jnp.int32, (LANES,), 0)

    pltpu.sync_copy(tb_hbm.at[pl.ds(qbase * LANES, QPW * LANES)], tb_v)
    pltpu.sync_copy(qbx_hbm.at[pl.ds(qbase * LANES, QPW * LANES)], qbx_v)
    pltpu.sync_copy(qby_hbm.at[pl.ds(qbase * LANES, QPW * LANES)], qby_v)
    pltpu.sync_copy(qbz_hbm.at[pl.ds(qbase * LANES, QPW * LANES)], qbz_v)
    pltpu.sync_copy(px_hbm.at[pl.ds(cloud * P, P)], px_v)
    pltpu.sync_copy(py_hbm.at[pl.ds(cloud * P, P)], py_v)
    pltpu.sync_copy(pz_hbm.at[pl.ds(cloud * P, P)], pz_v)
    pltpu.make_async_copy(d2_hbm.at[pl.ds(qbase * P, P)], d2a_v, sema).start()

    def process_one(i, j, d2buf):
        tv = plsc.load_gather(tb_v, [i * LANES + iota])

        def step(s, carry):
            cnt_m1, iv = carry
            d2v = plsc.load_gather(d2buf, [iv])
            m = d2v <= tv
            ranks = plsc.cumsum(jnp.where(m, 1, 0))
            tgt = cnt_m1 + ranks
            keep = jnp.logical_and(m, tgt <= K - 1)
            plsc.store_scatter(idx_v, [tgt + (j * K)], iv, mask=keep)
            pc = plsc.all_reduce_population_count(keep)
            return (cnt_m1 + pc, iv + LANES)

        cnt_m1, _ = jax.lax.fori_loop(
            0, STEPS, step,
            (jnp.full((LANES,), -1, jnp.int32), iota), unroll=8)
        v0 = idx_v[pl.ds(j * K, LANES)]
        pad = plsc.cummax(jnp.where(iota == 0, v0, jnp.int32(-(2 ** 31))))
        for k in range(K // LANES):
            lanes = iota + (16 * k)
            plsc.store_scatter(idx_v, [lanes + (j * K)], pad,
                               mask=lanes > cnt_m1)
        qxv = plsc.load_gather(qbx_v, [i * LANES + iota])
        qyv = plsc.load_gather(qby_v, [i * LANES + iota])
        qzv = plsc.load_gather(qbz_v, [i * LANES + iota])
        for k in range(K // LANES):
            iv = idx_v[pl.ds(j * K + 16 * k, LANES)]
            rbx_v[pl.ds(j * K + 16 * k, LANES)] = (
                plsc.load_gather(px_v, [iv]) - qxv)
            rby_v[pl.ds(j * K + 16 * k, LANES)] = (
                plsc.load_gather(py_v, [iv]) - qyv)
            rbz_v[pl.ds(j * K + 16 * k, LANES)] = (
                plsc.load_gather(pz_v, [iv]) - qzv)
        if with_x:
            for k in range(K // LANES):
                iv = idx_v[pl.ds(j * K + 16 * k, LANES)]
                gidx_v[pl.ds(16 * k, LANES)] = iv + cloud * P
            gdesc = pltpu.make_async_copy(x1_hbm.at[gidx_v], xrow_v, semg)
            gdesc.start()
            gdesc.wait()
            pltpu.sync_copy(
                xrow_v, nbrx_hbm.at[pl.ds((qbase + i) * K, K)])

    nbatch = QPW // QB

    def batch_body(b, _):
        for j in range(QB):
            i = b * QB + j
            buf_cur = d2a_v if j % 2 == 0 else d2b_v
            buf_nxt = d2b_v if j % 2 == 0 else d2a_v
            sem_cur = sema if j % 2 == 0 else semb
            sem_nxt = semb if j % 2 == 0 else sema
            pltpu.make_async_copy(
                d2_hbm.at[pl.ds(qbase * P, P)], buf_cur, sem_cur).wait()

            @pl.when(i + 1 < QPW)
            def _():
                pltpu.make_async_copy(
                    d2_hbm.at[pl.ds((qbase + i + 1) * P, P)],
                    buf_nxt, sem_nxt).start()

            process_one(i, j, buf_cur)
        base = (qbase + b * QB) * K
        pltpu.sync_copy(rbx_v, rx_hbm.at[pl.ds(base, QB * K)])
        pltpu.sync_copy(rby_v, ry_hbm.at[pl.ds(base, QB * K)])
        pltpu.sync_copy(rbz_v, rz_hbm.at[pl.ds(base, QB * K)])
        return 0

    jax.lax.fori_loop(0, nbatch, batch_body, 0)


def _compact_pallas(d2, tb, qbx, qby, qbz, px, py, pz, x1=None):
    NQ, P = d2.shape
    S1 = NQ // B
    QPW = NQ // NWORKERS
    with_x = x1 is not None
    mesh = plsc.VectorSubcoreMesh(core_axis_name="c", subcore_axis_name="s")
    relshape = jax.ShapeDtypeStruct((NQ * K,), jnp.float32)
    out_type = [relshape, relshape, relshape]
    scratch = [
        pltpu.VMEM((P,), jnp.float32),
        pltpu.VMEM((P,), jnp.float32),
        pltpu.VMEM((QPW * LANES,), jnp.float32),
        pltpu.VMEM((QPW * LANES,), jnp.float32),
        pltpu.VMEM((QPW * LANES,), jnp.float32),
        pltpu.VMEM((QPW * LANES,), jnp.float32),
        pltpu.VMEM((P,), jnp.float32),
        pltpu.VMEM((P,), jnp.float32),
        pltpu.VMEM((P,), jnp.float32),
        pltpu.VMEM((16 * K,), jnp.int32),
        pltpu.VMEM((16 * K,), jnp.float32),
        pltpu.VMEM((16 * K,), jnp.float32),
        pltpu.VMEM((16 * K,), jnp.float32),
    ]
    if with_x:
        XF = x1.shape[1]
        out_type = out_type + [
            jax.ShapeDtypeStruct((NQ * K, XF), jnp.float32)]
        scratch = scratch + [
            pltpu.VMEM((K,), jnp.int32),
            pltpu.VMEM((K, XF), jnp.float32),
            pltpu.SemaphoreType.DMA,
        ]
    scratch = scratch + [pltpu.SemaphoreType.DMA, pltpu.SemaphoreType.DMA]

    def body(*refs):
        _compact_kernel(P, QPW, S1, with_x, refs)

    kfn = pl.kernel(
        body,
        out_type=out_type,
        mesh=mesh,
        compiler_params=pltpu.CompilerParams(needs_layout_passes=False),
        scratch_types=scratch,
    )
    args = [d2.reshape(NQ * P), tb.reshape(NQ * LANES),
            qbx.reshape(NQ * LANES), qby.reshape(NQ * LANES),
            qbz.reshape(NQ * LANES),
            px.reshape(B * P), py.reshape(B * P), pz.reshape(B * P)]
    if with_x:
        args.append(x1)
        rx, ry, rz, nbrx = kfn(*args)
        return (rx.reshape(NQ, K), ry.reshape(NQ, K), rz.reshape(NQ, K),
                nbrx)
    rx, ry, rz = kfn(*args)
    return rx.reshape(NQ, K), ry.reshape(NQ, K), rz.reshape(NQ, K), None


def _mlp1_kernel(rx_ref, ry_ref, rz_ref, w1_ref, b1_ref, w2_ref, b2_ref,
                 out_ref):
    QB = rx_ref.shape[0]
    F1 = w1_ref.shape[1]
    F2 = out_ref.shape[1]

    def pairs(ref):
        v = ref[...][:, :, None]
        return jnp.broadcast_to(v, (QB, K, F1)).reshape(QB * K, F1)

    h = (pairs(rx_ref) * w1_ref[0:1, :] + pairs(ry_ref) * w1_ref[1:2, :]
         + pairs(rz_ref) * w1_ref[2:3, :] + b1_ref[...])
    h = jnp.maximum(h, 0.0)
    h = jnp.maximum(h @ w2_ref[...] + b2_ref[...], 0.0)
    x = jnp.max(h.reshape(QB, K, w2_ref.shape[1]), axis=1)
    out_ref[...] = jnp.concatenate(
        [x, jnp.zeros((QB, F2 - x.shape[1]), jnp.float32)], axis=1)


def _mlp1_pallas(rx, ry, rz, W11, b11, W12, b12):
    NQ = rx.shape[0]
    QB = 128
    grid = NQ // QB
    rspec = pl.BlockSpec((QB, K), lambda g: (g, 0))
    F1 = W12.shape[0]
    F2 = 2 * W12.shape[1]
    x1 = pl.pallas_call(
        _mlp1_kernel,
        grid=(grid,),
        in_specs=[rspec, rspec, rspec,
                  pl.BlockSpec((3, F1), lambda g: (0, 0)),
                  pl.BlockSpec((1, F1), lambda g: (0, 0)),
                  pl.BlockSpec((F1, W12.shape[1]), lambda g: (0, 0)),
                  pl.BlockSpec((1, W12.shape[1]), lambda g: (0, 0))],
        out_specs=pl.BlockSpec((QB, F2), lambda g: (g, 0)),
        out_shape=jax.ShapeDtypeStruct((NQ, F2), jnp.float32),
    )(rx, ry, rz, W11, b11[None, :], W12, b12[None, :])
    return x1


def _mlp2g_kernel(nbrx_ref, rx_ref, ry_ref, rz_ref, qx_ref, qy_ref, qz_ref,
                  w21a_ref, w21r_ref, b21_ref, w22_ref, b22_ref,
                  wg1_ref, bg1_ref, wg2_ref, bg2_ref, out_ref, *, bpc):
    QB = rx_ref.shape[0]
    F1 = w21a_ref.shape[1]

    def pairs(ref, width):
        v = ref[...][:, :, None]
        return jnp.broadcast_to(v, (QB, K, width)).reshape(QB * K, width)

    h = (nbrx_ref[...] @ w21a_ref[...]
         + pairs(rx_ref, F1) * w21r_ref[0:1, :]
         + pairs(ry_ref, F1) * w21r_ref[1:2, :]
         + pairs(rz_ref, F1) * w21r_ref[2:3, :]
         + b21_ref[...])
    h = jnp.maximum(h, 0.0)
    h = jnp.maximum(h @ w22_ref[...] + b22_ref[...], 0.0)
    F2 = w22_ref.shape[1]
    x2 = jnp.max(h.reshape(QB, K, F2), axis=1)
    g = (x2 @ wg1_ref[0:F2, :]
         + qx_ref[...] * wg1_ref[F2:F2 + 1, :]
         + qy_ref[...] * wg1_ref[F2 + 1:F2 + 2, :]
         + qz_ref[...] * wg1_ref[F2 + 2:F2 + 3, :]
         + bg1_ref[...])
    g = jnp.maximum(g, 0.0)
    g = g @ wg2_ref[...] + bg2_ref[...]
    m = jnp.max(g, axis=0, keepdims=True)[None]
    gi = pl.program_id(0)

    @pl.when(gi % bpc == 0)
    def _():
        out_ref[...] = m

    @pl.when(gi % bpc != 0)
    def _():
        out_ref[...] = jnp.maximum(out_ref[...], m)


def _mlp2g_pallas(nbrx, rx, ry, rz, q2x, q2y, q2z,
                  W21a, W21r, b21, W22, b22, Wg1, bg1, Wg2, bg2):
    NQ = rx.shape[0]
    S2 = NQ // B
    QB = 64
    grid = NQ // QB
    bpc = S2 // QB
    rspec = pl.BlockSpec((QB, K), lambda g: (g, 0))
    qspec = pl.BlockSpec((QB, 1), lambda g: (g, 0))
    XF = W21a.shape[0]
    F1 = W21a.shape[1]
    F2 = W22.shape[1]
    G1 = Wg1.shape[1]
    G2 = Wg2.shape[1]
    out = pl.pallas_call(
        functools.partial(_mlp2g_kernel, bpc=bpc),
        grid=(grid,),
        in_specs=[pl.BlockSpec((QB * K, XF), lambda g: (g, 0)),
                  rspec, rspec, rspec, qspec, qspec, qspec,
                  pl.BlockSpec((XF, F1), lambda g: (0, 0)),
                  pl.BlockSpec((3, F1), lambda g: (0, 0)),
                  pl.BlockSpec((1, F1), lambda g: (0, 0)),
                  pl.BlockSpec((F1, F2), lambda g: (0, 0)),
                  pl.BlockSpec((1, F2), lambda g: (0, 0)),
                  pl.BlockSpec((F2 + 3, G1), lambda g: (0, 0)),
                  pl.BlockSpec((1, G1), lambda g: (0, 0)),
                  pl.BlockSpec((G1, G2), lambda g: (0, 0)),
                  pl.BlockSpec((1, G2), lambda g: (0, 0))],
        out_specs=pl.BlockSpec((1, 1, G2), lambda g: (g // bpc, 0, 0)),
        out_shape=jax.ShapeDtypeStruct((B, 1, G2), jnp.float32),
    )(nbrx, rx, ry, rz, q2x.reshape(NQ, 1), q2y.reshape(NQ, 1),
      q2z.reshape(NQ, 1), W21a, W21r, b21[None, :], W22, b22[None, :],
      Wg1, bg1[None, :], Wg2, bg2[None, :])
    return out.reshape(B, G2)


def kernel(pos, batch, W11, b11, W12, b12, W21, b21, W22, b22, Wg1, bg1, Wg2, bg2):
    pos3 = pos.reshape(B, N, 3)
    px, py, pz = pos3[:, :, 0], pos3[:, :, 1], pos3[:, :, 2]
    W21a = jnp.concatenate([W21[:K], jnp.zeros_like(W21[:K])], axis=0)
    W21r = W21[K:]
    qx, qy, qz = _fps_pallas(px, py, pz, N // 2)
    d2, tb, qbx, qby, qbz = _bisect_pallas(qx, qy, qz, px, py, pz, 0.2)
    rx, ry, rz, _ = _compact_pallas(d2, tb, qbx, qby, qbz, px, py, pz)
    x1 = _mlp1_pallas(rx, ry, rz, W11, b11, W12, b12)
    q2x, q2y, q2z = _fps_pallas(qx, qy, qz, N // 8)
    d2b, tbb, qbx2, qby2, qbz2 = _bisect_pallas(q2x, q2y, q2z, qx, qy, qz, 0.4)
    rx2, ry2, rz2, nbrx = _compact_pallas(
        d2b, tbb, qbx2, qby2, qbz2, qx, qy, qz, x1)
    return _mlp2g_pallas(nbrx, rx2, ry2, rz2, q2x, q2y, q2z,
                         W21a, W21r, b21, W22, b22, Wg1, bg1, Wg2, bg2)
